# baseline (speedup 1.0000x reference)
import jax
import jax.numpy as jnp
from jax import lax
from jax.experimental import pallas as pl
from jax.experimental.pallas import tpu as pltpu
from jax.experimental.pallas import tpu_sc as plsc

DTIME_MAX = 10.0

NC = 2
NS = 16
NW = NC * NS
LANES = 16

E_DIM = 100
HEAD = 20
RB = 400
PB = 80
NG = RB // LANES
SENT = HEAD


def _sc_body(uh_hbm, th_hbm, eh_hbm, rate_hbm, u2_hbm, t2_hbm, e2_hbm, out_hbm,
             u_v, t_v, rate_v, km_v, idx_v, val_v,
             u2_v, t2_v, e2_v, cnt_s, sem):
    N = u2_hbm.shape[0]
    RW = N // NW
    NB = RW // RB

    wid = lax.axis_index("s") * NC + lax.axis_index("c")
    base_w = wid * RW

    lane_iota = lax.iota(jnp.int32, LANES)

    def block_body(b, _):
        r0 = base_w + b * RB
        pltpu.sync_copy(rate_hbm.at[pl.ds(r0, RB)], rate_v)
        pltpu.sync_copy(uh_hbm.at[pl.ds(r0 * HEAD, RB * HEAD)], u_v)
        pltpu.sync_copy(th_hbm.at[pl.ds(r0 * HEAD, RB * HEAD)], t_v)
        cnt_s[0] = 0

        def group_body(g, _):
            g16 = g * LANES
            rows_l = g16 + lane_iota
            rate_g = rate_v[pl.ds(g16, LANES)]
            base = rows_l * HEAD
            km = jnp.full((LANES,), SENT, jnp.int32)
            for e in range(HEAD):
                uc = plsc.load_gather(u_v, [base + e])
                tc = plsc.load_gather(t_v, [base + e])
                crit = uc * rate_g / tc
                acc = crit < 1.0
                km = jnp.minimum(
                    km, jnp.where(acc,
                                  jnp.full((LANES,), e, jnp.int32),
                                  jnp.full((LANES,), SENT, jnp.int32)))
            km_v[pl.ds(g16, LANES)] = km
            idx_v[pl.ds(g16, LANES)] = (r0 + rows_l) * HEAD + jnp.minimum(
                km, jnp.full((LANES,), HEAD - 1, jnp.int32))
            cnt_s[0] = cnt_s[0] + jnp.sum((km >= SENT).astype(jnp.int32))
            return _

        lax.fori_loop(0, NG, group_body, None)

        pltpu.async_copy(eh_hbm.at[idx_v], val_v, sem).wait()

        @pl.when(cnt_s[0] > 0)
        def _phase2():
            for sub in range(RB // PB):
                s0 = sub * PB

                pltpu.sync_copy(u2_hbm.at[pl.ds(r0 + s0, PB), :], u2_v)
                pltpu.sync_copy(t2_hbm.at[pl.ds(r0 + s0, PB), :], t2_v)
                pltpu.sync_copy(e2_hbm.at[pl.ds(r0 + s0, PB), :], e2_v)

                def g2_body(g2, _):
                    g16 = g2 * LANES
                    rows16 = g16 + lane_iota
                    km_g = km_v[pl.ds(s0 + g16, LANES)]
                    unres = km_g >= SENT
                    rate_g = rate_v[pl.ds(s0 + g16, LANES)]

                    def col_body(e, km2):
                        col = jnp.broadcast_to(e, (LANES,))
                        uc = plsc.load_gather(u2_v, [rows16, col])
                        tc = plsc.load_gather(t2_v, [rows16, col])
                        crit = uc * rate_g / tc
                        acc = crit < 1.0
                        return jnp.minimum(
                            km2, jnp.where(acc, col,
                                           jnp.full((LANES,), E_DIM, jnp.int32)))

                    km2 = lax.fori_loop(
                        HEAD, E_DIM, col_body,
                        jnp.full((LANES,), E_DIM, jnp.int32))
                    found2 = km2 < E_DIM
                    val2 = plsc.load_gather(
                        e2_v, [rows16,
                               jnp.minimum(km2, jnp.full((LANES,), E_DIM - 1,
                                                         jnp.int32))])
                    vg = val_v[pl.ds(s0 + g16, LANES)]
                    val_v[pl.ds(s0 + g16, LANES)] = jnp.where(
                        unres,
                        jnp.where(found2, val2,
                                  jnp.full((LANES,), DTIME_MAX, jnp.float32)),
                        vg)
                    return _
                lax.fori_loop(0, PB // LANES, g2_body, None)

        pltpu.sync_copy(val_v, out_hbm.at[pl.ds(r0, RB)])
        return _

    lax.fori_loop(0, NB, block_body, None)


def kernel(unif_numbers, sample_rate, total_intensities, exp_numbers):
    B, L, S, E = unif_numbers.shape
    N = B * L * S
    u2 = unif_numbers.reshape(N, E)
    t2 = total_intensities.reshape(N, E)
    e2 = exp_numbers.reshape(N, E)
    one = sample_rate[0, 0] * 0.0 + 1.0
    uh = (u2[:, :HEAD] * one).reshape(N * HEAD)
    th = (t2[:, :HEAD] * one).reshape(N * HEAD)
    eh = (e2[:, :HEAD] * one).reshape(N * HEAD)
    r1 = jnp.broadcast_to(sample_rate.reshape(B * L, 1), (B * L, S)).reshape(N)

    mesh = plsc.VectorSubcoreMesh(core_axis_name="c", subcore_axis_name="s")
    run = pl.kernel(
        _sc_body,
        out_type=jax.ShapeDtypeStruct((N,), jnp.float32),
        mesh=mesh,
        scratch_types=[
            pltpu.VMEM((RB * HEAD,), jnp.float32),
            pltpu.VMEM((RB * HEAD,), jnp.float32),
            pltpu.VMEM((RB,), jnp.float32),
            pltpu.VMEM((RB,), jnp.int32),
            pltpu.VMEM((RB,), jnp.int32),
            pltpu.VMEM((RB,), jnp.float32),
            pltpu.VMEM((PB, E_DIM), jnp.float32),
            pltpu.VMEM((PB, E_DIM), jnp.float32),
            pltpu.VMEM((PB, E_DIM), jnp.float32),
            pltpu.SMEM((1,), jnp.int32),
            pltpu.SemaphoreType.DMA,
        ],
        compiler_params=pltpu.CompilerParams(needs_layout_passes=False),
    )
    out = run(uh, th, eh, r1, u2, t2, e2)
    return out.reshape(B, L, S)


# speedup vs baseline: 1.6359x; 1.6359x over previous
import jax
import jax.numpy as jnp
from jax import lax
from jax.experimental import pallas as pl
from jax.experimental.pallas import tpu as pltpu
from jax.experimental.pallas import tpu_sc as plsc

DTIME_MAX = 10.0

NC = 2
NS = 16
NW = NC * NS
LANES = 16

E_DIM = 100
RB = 80
NG = RB // LANES
CH = 4
NCH = E_DIM // CH


def _sc_body(u_hbm, t_hbm, e_hbm, rate_hbm, out_hbm,
             u0_v, t0_v, e0_v, u1_v, t1_v, e1_v,
             rate_v, out_v, sem0, sem1):
    N = u_hbm.shape[0]
    RW = N // NW
    NB = RW // RB

    wid = lax.axis_index("s") * NC + lax.axis_index("c")
    base_w = wid * RW

    lane_iota = lax.iota(jnp.int32, LANES)

    pltpu.sync_copy(rate_hbm.at[pl.ds(base_w, RW)], rate_v)

    def issue(b, uv, tv, ev, sem):
        r0 = base_w + b * RB
        pltpu.make_async_copy(u_hbm.at[pl.ds(r0, RB), :], uv, sem).start()
        pltpu.make_async_copy(t_hbm.at[pl.ds(r0, RB), :], tv, sem).start()
        pltpu.make_async_copy(e_hbm.at[pl.ds(r0, RB), :], ev, sem).start()

    def drain(uv, tv, ev, sem):
        pltpu.make_async_copy(u_hbm.at[pl.ds(0, RB), :], uv, sem).wait()
        pltpu.make_async_copy(t_hbm.at[pl.ds(0, RB), :], tv, sem).wait()
        pltpu.make_async_copy(e_hbm.at[pl.ds(0, RB), :], ev, sem).wait()

    def compute(b, uv, tv, ev):
        ob = b * RB

        def group_body(g, _):
            g16 = g * LANES
            rows16 = g16 + lane_iota
            rate_g = rate_v[pl.ds(ob + g16, LANES)]

            def cond(carry):
                i, km = carry
                return (i < NCH) & (jnp.max(km) >= E_DIM)

            def body(carry):
                i, km = carry
                e0 = i * CH
                for j in range(CH):
                    col = jnp.broadcast_to(e0 + j, (LANES,))
                    uc = plsc.load_gather(uv, [rows16, col])
                    tc = plsc.load_gather(tv, [rows16, col])
                    crit = uc * rate_g / tc
                    acc = crit < 1.0
                    km = jnp.minimum(
                        km, jnp.where(acc, col,
                                      jnp.full((LANES,), E_DIM, jnp.int32)))
                return i + 1, km

            km = lax.while_loop(
                cond, body,
                (jnp.int32(0), jnp.full((LANES,), E_DIM, jnp.int32)))[1]

            val = plsc.load_gather(
                ev, [rows16,
                     jnp.minimum(km, jnp.full((LANES,), E_DIM - 1, jnp.int32))])
            out_v[pl.ds(ob + g16, LANES)] = jnp.where(
                km >= E_DIM, jnp.full((LANES,), DTIME_MAX, jnp.float32), val)
            return _

        lax.fori_loop(0, NG, group_body, None)

    HALF = NB // 2
    issue(0, u0_v, t0_v, e0_v, sem0)

    def pair_body(p, _):
        b0 = 2 * p
        drain(u0_v, t0_v, e0_v, sem0)
        issue(b0 + 1, u1_v, t1_v, e1_v, sem1)
        compute(b0, u0_v, t0_v, e0_v)
        drain(u1_v, t1_v, e1_v, sem1)

        @pl.when(b0 + 2 < NB)
        def _issue_next():
            issue(b0 + 2, u0_v, t0_v, e0_v, sem0)

        compute(b0 + 1, u1_v, t1_v, e1_v)
        return None

    lax.fori_loop(0, HALF, pair_body, None)
    drain(u0_v, t0_v, e0_v, sem0)
    compute(NB - 1, u0_v, t0_v, e0_v)

    pltpu.sync_copy(out_v, out_hbm.at[pl.ds(base_w, RW)])


def kernel(unif_numbers, sample_rate, total_intensities, exp_numbers):
    B, L, S, E = unif_numbers.shape
    N = B * L * S
    RW = N // NW
    u2 = unif_numbers.reshape(N, E)
    t2 = total_intensities.reshape(N, E)
    e2 = exp_numbers.reshape(N, E)
    r1 = jnp.broadcast_to(sample_rate.reshape(B * L, 1), (B * L, S)).reshape(N)

    mesh = plsc.VectorSubcoreMesh(core_axis_name="c", subcore_axis_name="s")
    run = pl.kernel(
        _sc_body,
        out_type=jax.ShapeDtypeStruct((N,), jnp.float32),
        mesh=mesh,
        scratch_types=[
            pltpu.VMEM((RB, E_DIM), jnp.float32),
            pltpu.VMEM((RB, E_DIM), jnp.float32),
            pltpu.VMEM((RB, E_DIM), jnp.float32),
            pltpu.VMEM((RB, E_DIM), jnp.float32),
            pltpu.VMEM((RB, E_DIM), jnp.float32),
            pltpu.VMEM((RB, E_DIM), jnp.float32),
            pltpu.VMEM((RW,), jnp.float32),
            pltpu.VMEM((RW,), jnp.float32),
            pltpu.SemaphoreType.DMA,
            pltpu.SemaphoreType.DMA,
        ],
        compiler_params=pltpu.CompilerParams(needs_layout_passes=False),
    )
    out = run(u2, t2, e2, r1)
    return out.reshape(B, L, S)


# speedup vs baseline: 9.0503x; 5.5324x over previous
import jax
import jax.numpy as jnp
from jax import lax
from jax.experimental import pallas as pl
from jax.experimental.pallas import tpu as pltpu
from jax.experimental.pallas import tpu_sc as plsc

DTIME_MAX = 10.0

NC = 2
NS = 16
NW = NC * NS
LANES = 16

E_DIM = 100
RB = 80
NG = RB // LANES
CH = 4
NCH = E_DIM // CH
NBUF = 4


def _sc_body(u_hbm, t_hbm, e_hbm, rate_hbm, out_hbm,
             u0_v, t0_v, e0_v, r0_v, u1_v, t1_v, e1_v, r1_v,
             u2_v, t2_v, e2_v, r2_v, u3_v, t3_v, e3_v, r3_v,
             out_q, sem0, sem1, sem2, sem3, osem):
    N = u_hbm.shape[0]
    RW = N // NW
    NB = RW // RB
    QR = NBUF * RB

    wid = lax.axis_index("s") * NC + lax.axis_index("c")
    base_w = wid * RW

    lane_iota = lax.iota(jnp.int32, LANES)

    bufs = ((u0_v, t0_v, e0_v, r0_v, sem0),
            (u1_v, t1_v, e1_v, r1_v, sem1),
            (u2_v, t2_v, e2_v, r2_v, sem2),
            (u3_v, t3_v, e3_v, r3_v, sem3))

    def issue(b, uv, tv, ev, rv, sem):
        r0 = base_w + b * RB
        pltpu.make_async_copy(u_hbm.at[pl.ds(r0, RB), :], uv, sem).start()
        pltpu.make_async_copy(t_hbm.at[pl.ds(r0, RB), :], tv, sem).start()
        pltpu.make_async_copy(e_hbm.at[pl.ds(r0, RB), :], ev, sem).start()
        pltpu.make_async_copy(rate_hbm.at[pl.ds(r0, RB)], rv, sem).start()

    def drain(uv, tv, ev, rv, sem):
        pltpu.make_async_copy(u_hbm.at[pl.ds(0, RB), :], uv, sem).wait()
        pltpu.make_async_copy(t_hbm.at[pl.ds(0, RB), :], tv, sem).wait()
        pltpu.make_async_copy(e_hbm.at[pl.ds(0, RB), :], ev, sem).wait()
        pltpu.make_async_copy(rate_hbm.at[pl.ds(0, RB)], rv, sem).wait()

    def compute(uv, tv, ev, rv, oloc):
        def group_body(g, _):
            g16 = g * LANES
            rows16 = g16 + lane_iota
            rate_g = rv[pl.ds(g16, LANES)]

            def cond(carry):
                i, km = carry
                return (i < NCH) & (jnp.max(km) >= E_DIM)

            def body(carry):
                i, km = carry
                e0 = i * CH
                for j in range(CH):
                    col = jnp.broadcast_to(e0 + j, (LANES,))
                    uc = plsc.load_gather(uv, [rows16, col])
                    tc = plsc.load_gather(tv, [rows16, col])
                    crit = uc * rate_g / tc
                    acc = crit < 1.0
                    km = jnp.minimum(
                        km, jnp.where(acc, col,
                                      jnp.full((LANES,), E_DIM, jnp.int32)))
                return i + 1, km

            km = lax.while_loop(
                cond, body,
                (jnp.int32(0), jnp.full((LANES,), E_DIM, jnp.int32)))[1]

            val = plsc.load_gather(
                ev, [rows16,
                     jnp.minimum(km, jnp.full((LANES,), E_DIM - 1, jnp.int32))])
            out_q[pl.ds(oloc + g16, LANES)] = jnp.where(
                km >= E_DIM, jnp.full((LANES,), DTIME_MAX, jnp.float32), val)
            return _

        lax.fori_loop(0, NG, group_body, None)

    NQ = NB // NBUF
    for k in range(NBUF):
        issue(k, *bufs[k])

    def out_wait():
        pltpu.make_async_copy(
            out_q, out_hbm.at[pl.ds(base_w, QR)], osem).wait()

    def quad_body(q, _):
        @pl.when(q > 0)
        def _wait_prev_out():
            out_wait()

        b0 = NBUF * q
        for k in range(NBUF):
            bk = b0 + k
            uv, tv, ev, rv, sem = bufs[k]
            drain(uv, tv, ev, rv, sem)
            compute(uv, tv, ev, rv, k * RB)

            @pl.when(bk + NBUF < NB)
            def _issue_next():
                issue(bk + NBUF, uv, tv, ev, rv, sem)

        pltpu.make_async_copy(
            out_q, out_hbm.at[pl.ds(base_w + b0 * RB, QR)], osem).start()
        return None

    lax.fori_loop(0, NQ, quad_body, None)

    out_wait()
    uv, tv, ev, rv, sem = bufs[0]
    drain(uv, tv, ev, rv, sem)
    compute(uv, tv, ev, rv, 0)
    pltpu.sync_copy(out_q.at[pl.ds(0, RB)],
                    out_hbm.at[pl.ds(base_w + (NB - 1) * RB, RB)])


def kernel(unif_numbers, sample_rate, total_intensities, exp_numbers):
    B, L, S, E = unif_numbers.shape
    N = B * L * S
    u2 = unif_numbers.transpose(0, 2, 1, 3).reshape(N, E)
    t2 = total_intensities.transpose(0, 2, 1, 3).reshape(N, E)
    e2 = exp_numbers.transpose(0, 2, 1, 3).reshape(N, E)
    r1 = jnp.broadcast_to(sample_rate[:, None, :], (B, S, L)).reshape(N)

    slab = pltpu.VMEM((RB, E_DIM), jnp.float32)
    rslab = pltpu.VMEM((RB,), jnp.float32)
    mesh = plsc.VectorSubcoreMesh(core_axis_name="c", subcore_axis_name="s")
    run = pl.kernel(
        _sc_body,
        out_type=jax.ShapeDtypeStruct((N,), jnp.float32),
        mesh=mesh,
        scratch_types=[
            slab, slab, slab, rslab,
            slab, slab, slab, rslab,
            slab, slab, slab, rslab,
            slab, slab, slab, rslab,
            pltpu.VMEM((NBUF * RB,), jnp.float32),
            pltpu.SemaphoreType.DMA,
            pltpu.SemaphoreType.DMA,
            pltpu.SemaphoreType.DMA,
            pltpu.SemaphoreType.DMA,
            pltpu.SemaphoreType.DMA,
        ],
        compiler_params=pltpu.CompilerParams(needs_layout_passes=False),
    )
    out = run(u2, t2, e2, r1)
    return out.reshape(B, S, L).transpose(0, 2, 1)
